# Initial kernel scaffold; baseline (speedup 1.0000x reference)
import jax
import jax.numpy as jnp
from jax.experimental import pallas as pl


def kernel(x, edge_index, W0, b0, W1, b1, W2, b2):
    raise NotImplementedError("write your pallas kernel here")


# speedup vs baseline: 3.5196x; 3.5196x over previous
import functools

import jax
import jax.numpy as jnp
from jax import lax
from jax.experimental import pallas as pl
from jax.experimental.pallas import tpu as pltpu
from jax.experimental.pallas import tpu_sc as plsc

N = 10000
E = 320000
D = 128
HALF_N = 5000
ACC_ROWS = 5120
TRASH = 5056
NEG_SLOPE = 0.01

CHUNK = 128
E_PAD = 327680
EPT = E_PAD // 16
KT = EPT // CHUNK
ROWS_PER_TILE = ACC_ROWS // 16

_mesh = plsc.VectorSubcoreMesh(core_axis_name="c", subcore_axis_name="s")


@functools.partial(
    pl.kernel,
    out_type=(
        jax.ShapeDtypeStruct((2, ACC_ROWS, D), jnp.float32),
        jax.ShapeDtypeStruct((2, 16, KT, CHUNK), jnp.int32),
    ),
    mesh=_mesh,
    scratch_types=[
        pltpu.VMEM((KT, CHUNK), jnp.int32),
        pltpu.VMEM((KT, CHUNK), jnp.int32),
        pltpu.VMEM((CHUNK, D), jnp.float32),
        pltpu.VMEM((CHUNK, D), jnp.float32),
        pltpu.VMEM_SHARED((ACC_ROWS, D), jnp.float32),
    ],
)
def _deg_dst2_kernel(src_hbm, dst_hbm, deg_out, dst2_out, sidx, didx, ones,
                     zeros, acc):
    c = lax.axis_index("c")
    s = lax.axis_index("s")
    base = c * HALF_N

    def _fill(i, _):
        for j in range(D // 16):
            ones[i, pl.ds(j * 16, 16)] = jnp.full((16,), 1.0, jnp.float32)
            zeros[i, pl.ds(j * 16, 16)] = jnp.zeros((16,), jnp.float32)
        return 0
    lax.fori_loop(0, CHUNK, _fill, 0)

    for j in range(ROWS_PER_TILE // CHUNK):
        pltpu.sync_copy(zeros, acc.at[pl.ds(s * ROWS_PER_TILE + j * CHUNK, CHUNK)])
    if ROWS_PER_TILE % CHUNK:
        r = ROWS_PER_TILE % CHUNK
        pltpu.sync_copy(
            zeros.at[pl.ds(0, r)],
            acc.at[pl.ds(s * ROWS_PER_TILE + ROWS_PER_TILE - r, r)])

    pltpu.sync_copy(src_hbm.at[s], sidx)
    pltpu.sync_copy(dst_hbm.at[s], didx)

    def _mask(k, _):
        for j in range(CHUNK // 16):
            sv = sidx[k, pl.ds(j * 16, 16)]
            dv = didx[k, pl.ds(j * 16, 16)]
            t = dv - base
            keep = (t >= 0) & (t < HALF_N) & (sv != dv)
            didx[k, pl.ds(j * 16, 16)] = jnp.where(keep, t, TRASH)
        return 0
    lax.fori_loop(0, KT, _mask, 0)

    pltpu.sync_copy(didx, dst2_out.at[c, s])

    plsc.subcore_barrier()

    def _scatter(k, _):
        pltpu.sync_copy(ones, acc.at[didx.at[k]], add=True)
        return 0
    lax.fori_loop(0, KT, _scatter, 0)

    plsc.subcore_barrier()

    sl = pl.ds(s * ROWS_PER_TILE, ROWS_PER_TILE)
    pltpu.sync_copy(acc.at[sl], deg_out.at[c, sl])


@functools.partial(
    pl.kernel,
    out_type=jax.ShapeDtypeStruct((2, ACC_ROWS, D), jnp.float32),
    mesh=_mesh,
    scratch_types=[
        pltpu.VMEM((KT, CHUNK), jnp.int32),
        pltpu.VMEM((KT, CHUNK), jnp.int32),
        pltpu.VMEM((CHUNK, D), jnp.float32),
        pltpu.VMEM((CHUNK, D), jnp.float32),
        pltpu.VMEM_SHARED((ACC_ROWS, D), jnp.float32),
        pltpu.SemaphoreType.DMA,
    ],
)
def _agg_kernel(y_hbm, src_hbm, dst_hbm, out_hbm, sidx, didx, rows, zbuf, acc,
                sem):
    c = lax.axis_index("c")
    s = lax.axis_index("s")

    def _zfill(i, _):
        for j in range(D // 16):
            zbuf[i, pl.ds(j * 16, 16)] = jnp.zeros((16,), jnp.float32)
        return 0
    lax.fori_loop(0, CHUNK, _zfill, 0)

    for j in range(ROWS_PER_TILE // CHUNK):
        pltpu.sync_copy(zbuf, acc.at[pl.ds(s * ROWS_PER_TILE + j * CHUNK, CHUNK)])
    if ROWS_PER_TILE % CHUNK:
        r = ROWS_PER_TILE % CHUNK
        pltpu.sync_copy(
            zbuf.at[pl.ds(0, r)],
            acc.at[pl.ds(s * ROWS_PER_TILE + ROWS_PER_TILE - r, r)])

    pltpu.sync_copy(src_hbm.at[s], sidx)
    pltpu.sync_copy(dst_hbm.at[c, s], didx)

    plsc.subcore_barrier()

    def _edge(k, _):
        pltpu.async_copy(y_hbm.at[sidx.at[k]], rows, sem).wait()
        pltpu.sync_copy(rows, acc.at[didx.at[k]], add=True)
        return 0
    lax.fori_loop(0, KT, _edge, 0)

    plsc.subcore_barrier()

    sl = pl.ds(s * ROWS_PER_TILE, ROWS_PER_TILE)
    pltpu.sync_copy(acc.at[sl], out_hbm.at[c, sl])


BN = 1000
GRID = N // BN


def _tcprep_body(deg_ref, dinv_ref):
    deg = deg_ref[0, :, 0] + 2.0
    dinv_ref[0, 0, :] = lax.rsqrt(deg)


def _tc0_body(dinv_ref, x_ref, w_ref, y_ref):
    xw = jnp.dot(x_ref[...], w_ref[...], preferred_element_type=jnp.float32)
    y_ref[...] = dinv_ref[0, 0, :][:, None] * xw


def _tcmid_body(dinv_ref, s_ref, yp_ref, w_ref, b_ref, y_ref):
    dinv = dinv_ref[0, 0, :]
    accf = s_ref[0] + 2.0 * yp_ref[...]
    h = dinv[:, None] * accf + b_ref[...]
    h = jnp.where(h > 0, h, NEG_SLOPE * h)
    y_ref[...] = dinv[:, None] * jnp.dot(h, w_ref[...],
                                         preferred_element_type=jnp.float32)


def _tcfin_body(dinv_ref, s_ref, yp_ref, b_ref, o_ref):
    dinv = dinv_ref[0, 0, :]
    accf = s_ref[0] + 2.0 * yp_ref[...]
    h = dinv[:, None] * accf + b_ref[...]
    o_ref[...] = jnp.where(h > 0, h, NEG_SLOPE * h)


_half_spec = pl.BlockSpec((1, BN, D), lambda i: (i // 5, i % 5, 0))
_dinv_spec = pl.BlockSpec((1, 1, BN), lambda i: (i, 0, 0))
_full_spec = pl.BlockSpec((BN, D), lambda i: (i, 0))
_w_spec = pl.BlockSpec((D, D), lambda i: (0, 0))
_b_spec = pl.BlockSpec((D,), lambda i: (0,))

_tcprep = pl.pallas_call(
    _tcprep_body,
    grid=(GRID,),
    in_specs=[_half_spec],
    out_specs=_dinv_spec,
    out_shape=jax.ShapeDtypeStruct((GRID, 1, BN), jnp.float32),
)

_tc0 = pl.pallas_call(
    _tc0_body,
    grid=(GRID,),
    in_specs=[_dinv_spec, _full_spec, _w_spec],
    out_specs=_full_spec,
    out_shape=jax.ShapeDtypeStruct((N, D), jnp.float32),
)

_tcmid = pl.pallas_call(
    _tcmid_body,
    grid=(GRID,),
    in_specs=[_dinv_spec, _half_spec, _full_spec, _w_spec, _b_spec],
    out_specs=_full_spec,
    out_shape=jax.ShapeDtypeStruct((N, D), jnp.float32),
)

_tcfin = pl.pallas_call(
    _tcfin_body,
    grid=(GRID,),
    in_specs=[_dinv_spec, _half_spec, _full_spec, _b_spec],
    out_specs=_full_spec,
    out_shape=jax.ShapeDtypeStruct((N, D), jnp.float32),
)


def kernel(x, edge_index, W0, b0, W1, b1, W2, b2):
    src = edge_index[0]
    dst = edge_index[1]
    pad = E_PAD - E
    src_s = jnp.concatenate([src, jnp.zeros((pad,), jnp.int32)]).reshape(
        16, KT, CHUNK)
    dst_s = jnp.concatenate([dst, jnp.full((pad,), N, jnp.int32)]).reshape(
        16, KT, CHUNK)

    deg, dst2 = _deg_dst2_kernel(src_s, dst_s)
    dinv = _tcprep(deg)

    y = _tc0(dinv, x, W0)
    s0 = _agg_kernel(y, src_s, dst2)
    y = _tcmid(dinv, s0, y, W1, b0)
    s1 = _agg_kernel(y, src_s, dst2)
    y = _tcmid(dinv, s1, y, W2, b1)
    s2 = _agg_kernel(y, src_s, dst2)
    return _tcfin(dinv, s2, y, b2)
